# baseline (speedup 1.0000x reference)
import functools

import jax
import jax.numpy as jnp
from jax import lax
from jax.experimental import pallas as pl
from jax.experimental.pallas import tpu as pltpu
from jax.experimental.pallas import tpu_sc as plsc

N_NODES = 10000
N_EDGES = 320000
D = 128
H = 64

_NODE_BLK = 1000


def _k1_body(z_ref, w1_ref, zv_ref, b1_ref, t_ref):
    z = z_ref[...]
    c = jnp.dot(zv_ref[...], w1_ref[2 * D:3 * D, :],
                preferred_element_type=jnp.float32) + b1_ref[...]
    a = jnp.dot(z, w1_ref[0:D, :], preferred_element_type=jnp.float32) + c
    b = jnp.dot(z, w1_ref[D:2 * D, :], preferred_element_type=jnp.float32)
    t_ref[...] = jnp.concatenate([a, b], axis=1)


def _build_table(Z, W1, zv, b1):
    n_blocks = N_NODES // _NODE_BLK
    return pl.pallas_call(
        _k1_body,
        grid=(n_blocks,),
        in_specs=[
            pl.BlockSpec((_NODE_BLK, D), lambda i: (i, 0)),
            pl.BlockSpec((3 * D, H), lambda i: (0, 0)),
            pl.BlockSpec((1, D), lambda i: (0, 0)),
            pl.BlockSpec((1, H), lambda i: (0, 0)),
        ],
        out_specs=pl.BlockSpec((_NODE_BLK, 2 * H), lambda i: (i, 0)),
        out_shape=jax.ShapeDtypeStruct((N_NODES, 2 * H), jnp.float32),
    )(Z, W1, zv, b1)


_NW = 32
_NSEG = 2
_ESEG = N_EDGES // _NSEG
_EPW = _ESEG // _NW
_CHUNK = 80
_NCHF = _EPW // _CHUNK
_TAIL = _EPW - _NCHF * _CHUNK


def _fuse_relu(ra_v, rb_v, ho_v, n):
    def fuse(e, c2):
        for j in range(H // 16):
            ho_v[e, pl.ds(j * 16, 16)] = jnp.maximum(
                ra_v[e, pl.ds(j * 16, 16)]
                + rb_v[e, pl.ds(H + j * 16, 16)], 0.0)
        return c2

    lax.fori_loop(0, n, fuse, 0, unroll=4)


def _make_k2_body(seg):
    def _k2_body(t_hbm, ei_hbm, out_hbm,
                 ia_v, ib_v, ra0_v, rb0_v, ra1_v, rb1_v, ra2_v, rb2_v,
                 ho0_v, ho1_v, ho2_v, sg0, sg1, sg2, so0, so1, so2):
        wid = lax.axis_index("s") * 2 + lax.axis_index("c")
        ebase = seg * _ESEG + wid * _EPW
        obase = wid * _EPW
        ra = [ra0_v, ra1_v, ra2_v]
        rb = [rb0_v, rb1_v, rb2_v]
        ho = [ho0_v, ho1_v, ho2_v]
        sg = [sg0, sg1, sg2]
        so = [so0, so1, so2]

        pltpu.sync_copy(ei_hbm.at[pl.ds(ebase, _EPW)], ia_v)
        pltpu.sync_copy(ei_hbm.at[pl.ds(N_EDGES + ebase, _EPW)], ib_v)

        def start_gather(c, s, n=_CHUNK):
            pltpu.async_copy(
                t_hbm.at[ia_v.at[pl.ds(c * _CHUNK, n)]],
                ra[s].at[pl.ds(0, n), :], sg[s])
            pltpu.async_copy(
                t_hbm.at[ib_v.at[pl.ds(c * _CHUNK, n)]],
                rb[s].at[pl.ds(0, n), :], sg[s])

        def wait_gather(s, n=_CHUNK):
            pltpu.make_async_copy(t_hbm.at[ia_v.at[pl.ds(0, n)]],
                                  ra[s].at[pl.ds(0, n), :], sg[s]).wait()
            pltpu.make_async_copy(t_hbm.at[ib_v.at[pl.ds(0, n)]],
                                  rb[s].at[pl.ds(0, n), :], sg[s]).wait()

        def start_out(c, s):
            pltpu.async_copy(
                ho[s], out_hbm.at[pl.ds(obase + c * _CHUNK, _CHUNK)], so[s])

        def wait_out(s):
            pltpu.make_async_copy(ho[s], out_hbm.at[pl.ds(0, _CHUNK)],
                                  so[s]).wait()

        start_gather(0, 0)
        start_gather(1, 1)

        def triple(i, carry):
            c0 = 3 * i
            for k in range(3):
                start_gather(c0 + k + 2, (k + 2) % 3)
                wait_gather(k)

                @pl.when(i > 0)
                def _():
                    wait_out(k)

                _fuse_relu(ra[k], rb[k], ho[k], _CHUNK)
                start_out(c0 + k, k)
            return carry

        n3 = (_NCHF - 2) // 3
        lax.fori_loop(0, n3, triple, 0)

        for c in range(3 * n3, _NCHF):
            s = c % 3
            if c >= 3 * n3 + 2:
                start_gather(c, s)
            wait_gather(s)
            wait_out(s)
            _fuse_relu(ra[s], rb[s], ho[s], _CHUNK)
            start_out(c, s)

        s = _NCHF % 3
        start_gather(_NCHF, s, _TAIL)
        wait_gather(s, _TAIL)
        wait_out(s)
        _fuse_relu(ra[s], rb[s], ho[s], _TAIL)
        pltpu.sync_copy(
            ho[s].at[pl.ds(0, _TAIL), :],
            out_hbm.at[pl.ds(obase + _NCHF * _CHUNK, _TAIL)])
        for k in range(3):
            if k != s:
                wait_out(k)

    return _k2_body


def _gather_relu(T, edge_index, seg):
    mesh = plsc.VectorSubcoreMesh(core_axis_name="c", subcore_axis_name="s")
    k = functools.partial(
        pl.kernel,
        mesh=mesh,
        out_type=jax.ShapeDtypeStruct((_ESEG, H), jnp.float32),
        scratch_types=(
            [pltpu.VMEM((_EPW,), jnp.int32)] * 2
            + [pltpu.VMEM((_CHUNK, 2 * H), jnp.float32)] * 6
            + [pltpu.VMEM((_CHUNK, H), jnp.float32)] * 3
            + [pltpu.SemaphoreType.DMA] * 6
        ),
    )(_make_k2_body(seg))
    return k(T, edge_index)


_EDGE_BLK = 16000


def _k3_body(h_ref, w2_ref, b2_ref, w3_ref, b3_ref, o_ref):
    h2t = lax.dot_general(w2_ref[...], h_ref[...], (((0,), (1,)), ((), ())),
                          preferred_element_type=jnp.float32)
    h2t = jnp.maximum(h2t + b2_ref[...], 0.0)
    o_ref[...] = lax.dot_general(w3_ref[...], h2t, (((0,), (0,)), ((), ())),
                                 preferred_element_type=jnp.float32) + b3_ref[0]


def _mlp_tail(H1, W2, b2, W3, b3):
    n_edges = H1.shape[0]
    n_blocks = n_edges // _EDGE_BLK
    nh = W2.shape[1]
    return pl.pallas_call(
        _k3_body,
        grid=(n_blocks,),
        in_specs=[
            pl.BlockSpec((_EDGE_BLK, H), lambda i: (i, 0)),
            pl.BlockSpec((H, nh), lambda i: (0, 0)),
            pl.BlockSpec((nh, 1), lambda i: (0, 0)),
            pl.BlockSpec((nh, 1), lambda i: (0, 0)),
            pl.BlockSpec(memory_space=pltpu.SMEM),
        ],
        out_specs=pl.BlockSpec((1, _EDGE_BLK), lambda i: (0, i)),
        out_shape=jax.ShapeDtypeStruct((1, n_edges), jnp.float32),
    )(H1, W2, b2.reshape(nh, 1), W3, b3)


def kernel(Z, edge_index, node_idx, W1, b1, W2, b2, W3, b3):
    zv = lax.dynamic_slice(Z, (node_idx, 0), (1, D))
    T = _build_table(Z, W1, zv, b1.reshape(1, H))
    ei = edge_index.astype(jnp.int32).reshape(2 * N_EDGES)
    outs = []
    for seg in range(_NSEG):
        H1 = _gather_relu(T, ei, seg)
        outs.append(_mlp_tail(H1, W2, b2, W3, b3))
    return jnp.concatenate(outs, axis=1).reshape(N_EDGES)


# speedup vs baseline: 1.0018x; 1.0018x over previous
import functools

import jax
import jax.numpy as jnp
from jax import lax
from jax.experimental import pallas as pl
from jax.experimental.pallas import tpu as pltpu
from jax.experimental.pallas import tpu_sc as plsc

N_NODES = 10000
N_EDGES = 320000
D = 128
H = 64

_NODE_BLK = 1000


def _k1_body(z_ref, w1_ref, zv_ref, b1_ref, t_ref):
    z = z_ref[...]
    c = jnp.dot(zv_ref[...], w1_ref[2 * D:3 * D, :],
                preferred_element_type=jnp.float32) + b1_ref[...]
    a = jnp.dot(z, w1_ref[0:D, :], preferred_element_type=jnp.float32) + c
    b = jnp.dot(z, w1_ref[D:2 * D, :], preferred_element_type=jnp.float32)
    t_ref[...] = jnp.concatenate([a, b], axis=1)


def _build_table(Z, W1, zv, b1):
    n_blocks = N_NODES // _NODE_BLK
    return pl.pallas_call(
        _k1_body,
        grid=(n_blocks,),
        in_specs=[
            pl.BlockSpec((_NODE_BLK, D), lambda i: (i, 0)),
            pl.BlockSpec((3 * D, H), lambda i: (0, 0)),
            pl.BlockSpec((1, D), lambda i: (0, 0)),
            pl.BlockSpec((1, H), lambda i: (0, 0)),
        ],
        out_specs=pl.BlockSpec((_NODE_BLK, 2 * H), lambda i: (i, 0)),
        out_shape=jax.ShapeDtypeStruct((N_NODES, 2 * H), jnp.float32),
    )(Z, W1, zv, b1)


_NW = 32
_NSEG = 2
_ESEG = N_EDGES // _NSEG
_EPW = _ESEG // _NW
_CHUNK = 80
_NCHF = _EPW // _CHUNK
_TAIL = _EPW - _NCHF * _CHUNK


def _fuse_relu(ra_v, rb_v, ho_v, n):
    def fuse(e, c2):
        for j in range(H // 16):
            ho_v[e, pl.ds(j * 16, 16)] = jnp.maximum(
                ra_v[e, pl.ds(j * 16, 16)]
                + rb_v[e, pl.ds(H + j * 16, 16)], 0.0)
        return c2

    lax.fori_loop(0, n, fuse, 0, unroll=4)


def _make_k2_body(seg):
    def _k2_body(t_hbm, ei_hbm, out_hbm,
                 ia_v, ib_v, ra0_v, rb0_v, ra1_v, rb1_v, ra2_v, rb2_v,
                 ho0_v, ho1_v, ho2_v, sg0, sg1, sg2, so0, so1, so2):
        wid = lax.axis_index("s") * 2 + lax.axis_index("c")
        ebase = seg * _ESEG + wid * _EPW
        obase = wid * _EPW
        ra = [ra0_v, ra1_v, ra2_v]
        rb = [rb0_v, rb1_v, rb2_v]
        ho = [ho0_v, ho1_v, ho2_v]
        sg = [sg0, sg1, sg2]
        so = [so0, so1, so2]

        pltpu.sync_copy(ei_hbm.at[pl.ds(ebase, _EPW)], ia_v)
        pltpu.sync_copy(ei_hbm.at[pl.ds(N_EDGES + ebase, _EPW)], ib_v)

        def start_gather(c, s, n=_CHUNK):
            pltpu.async_copy(
                t_hbm.at[ia_v.at[pl.ds(c * _CHUNK, n)]],
                ra[s].at[pl.ds(0, n), :], sg[s])
            pltpu.async_copy(
                t_hbm.at[ib_v.at[pl.ds(c * _CHUNK, n)]],
                rb[s].at[pl.ds(0, n), :], sg[s])

        def wait_gather(s, n=_CHUNK):
            pltpu.make_async_copy(t_hbm.at[ia_v.at[pl.ds(0, n)]],
                                  ra[s].at[pl.ds(0, n), :], sg[s]).wait()
            pltpu.make_async_copy(t_hbm.at[ib_v.at[pl.ds(0, n)]],
                                  rb[s].at[pl.ds(0, n), :], sg[s]).wait()

        def start_out(c, s):
            pltpu.async_copy(
                ho[s], out_hbm.at[pl.ds(obase + c * _CHUNK, _CHUNK)], so[s])

        def wait_out(s):
            pltpu.make_async_copy(ho[s], out_hbm.at[pl.ds(0, _CHUNK)],
                                  so[s]).wait()

        start_gather(0, 0)
        start_gather(1, 1)

        def triple(i, carry):
            c0 = 3 * i
            for k in range(3):
                start_gather(c0 + k + 2, (k + 2) % 3)
                wait_gather(k)

                @pl.when(i > 0)
                def _():
                    wait_out(k)

                _fuse_relu(ra[k], rb[k], ho[k], _CHUNK)
                start_out(c0 + k, k)
            return carry

        n3 = (_NCHF - 2) // 3
        lax.fori_loop(0, n3, triple, 0)

        for c in range(3 * n3, _NCHF):
            s = c % 3
            if c >= 3 * n3 + 2:
                start_gather(c, s)
            wait_gather(s)
            wait_out(s)
            _fuse_relu(ra[s], rb[s], ho[s], _CHUNK)
            start_out(c, s)

        s = _NCHF % 3
        start_gather(_NCHF, s, _TAIL)
        wait_gather(s, _TAIL)
        wait_out(s)
        _fuse_relu(ra[s], rb[s], ho[s], _TAIL)
        pltpu.sync_copy(
            ho[s].at[pl.ds(0, _TAIL), :],
            out_hbm.at[pl.ds(obase + _NCHF * _CHUNK, _TAIL)])
        for k in range(3):
            if k != s:
                wait_out(k)

    return _k2_body


def _gather_relu(T, edge_index, seg):
    mesh = plsc.VectorSubcoreMesh(core_axis_name="c", subcore_axis_name="s")
    k = functools.partial(
        pl.kernel,
        mesh=mesh,
        out_type=jax.ShapeDtypeStruct((_ESEG, H), jnp.float32),
        scratch_types=(
            [pltpu.VMEM((_EPW,), jnp.int32)] * 2
            + [pltpu.VMEM((_CHUNK, 2 * H), jnp.float32)] * 6
            + [pltpu.VMEM((_CHUNK, H), jnp.float32)] * 3
            + [pltpu.SemaphoreType.DMA] * 6
        ),
    )(_make_k2_body(seg))
    return k(T, edge_index)


_EDGE_BLK = 16000


def _k3_body(h_ref, w2_ref, b2_ref, w3_ref, b3_ref, o_ref):
    h2t = lax.dot_general(w2_ref[...], h_ref[...], (((0,), (1,)), ((), ())),
                          preferred_element_type=jnp.float32)
    h2t = jnp.maximum(h2t + b2_ref[...], 0.0)
    o_ref[...] = lax.dot_general(w3_ref[...], h2t, (((0,), (0,)), ((), ())),
                                 preferred_element_type=jnp.float32) + b3_ref[0]


def _k3_body_seg(h_ref, w2_ref, b2_ref, w3_ref, b3_ref, prev_ref, o_ref):
    del prev_ref
    _k3_body(h_ref, w2_ref, b2_ref, w3_ref, b3_ref, o_ref)


def _mlp_tail(H1, W2, b2, W3, b3, seg, prev):
    n_edges = H1.shape[0]
    n_blocks = n_edges // _EDGE_BLK
    nh = W2.shape[1]
    base = seg * n_blocks
    if prev is None:
        body, extra, aliases = _k3_body, (), {}
    else:
        body, extra = _k3_body_seg, (prev,)
        aliases = {5: 0}
    return pl.pallas_call(
        body,
        grid=(n_blocks,),
        in_specs=[
            pl.BlockSpec((_EDGE_BLK, H), lambda i: (i, 0)),
            pl.BlockSpec((H, nh), lambda i: (0, 0)),
            pl.BlockSpec((nh, 1), lambda i: (0, 0)),
            pl.BlockSpec((nh, 1), lambda i: (0, 0)),
            pl.BlockSpec(memory_space=pltpu.SMEM),
        ] + ([pl.BlockSpec(memory_space=pl.ANY)] if prev is not None
             else []),
        out_specs=pl.BlockSpec((1, _EDGE_BLK), lambda i, base=base: (0, base + i)),
        out_shape=jax.ShapeDtypeStruct((1, N_EDGES), jnp.float32),
        input_output_aliases=aliases,
    )(H1, W2, b2.reshape(nh, 1), W3, b3, *extra)


def kernel(Z, edge_index, node_idx, W1, b1, W2, b2, W3, b3):
    zv = lax.dynamic_slice(Z, (node_idx, 0), (1, D))
    T = _build_table(Z, W1, zv, b1.reshape(1, H))
    ei = edge_index.astype(jnp.int32).reshape(2 * N_EDGES)
    out = None
    for seg in range(_NSEG):
        H1 = _gather_relu(T, ei, seg)
        out = _mlp_tail(H1, W2, b2, W3, b3, seg, out)
    return out.reshape(N_EDGES)


# speedup vs baseline: 1.0174x; 1.0156x over previous
import functools

import jax
import jax.numpy as jnp
from jax import lax
from jax.experimental import pallas as pl
from jax.experimental.pallas import tpu as pltpu
from jax.experimental.pallas import tpu_sc as plsc

N_NODES = 10000
N_EDGES = 320000
D = 128
H = 64

_NODE_BLK = 1000


def _k1_body(z_ref, w1_ref, zv_ref, b1_ref, t_ref):
    z = z_ref[...]
    c = jnp.dot(zv_ref[...], w1_ref[2 * D:3 * D, :],
                preferred_element_type=jnp.float32) + b1_ref[...]
    a = jnp.dot(z, w1_ref[0:D, :], preferred_element_type=jnp.float32) + c
    b = jnp.dot(z, w1_ref[D:2 * D, :], preferred_element_type=jnp.float32)
    t_ref[...] = jnp.concatenate([a, b], axis=1)


def _build_table(Z, W1, zv, b1):
    n_blocks = N_NODES // _NODE_BLK
    return pl.pallas_call(
        _k1_body,
        grid=(n_blocks,),
        in_specs=[
            pl.BlockSpec((_NODE_BLK, D), lambda i: (i, 0)),
            pl.BlockSpec((3 * D, H), lambda i: (0, 0)),
            pl.BlockSpec((1, D), lambda i: (0, 0)),
            pl.BlockSpec((1, H), lambda i: (0, 0)),
        ],
        out_specs=pl.BlockSpec((_NODE_BLK, 2 * H), lambda i: (i, 0)),
        out_shape=jax.ShapeDtypeStruct((N_NODES, 2 * H), jnp.float32),
    )(Z, W1, zv, b1)


_NW = 32
_NSEG = 2
_ESEG = N_EDGES // _NSEG
_EPW = _ESEG // _NW
_CHUNK = 40
_NCHF = _EPW // _CHUNK
_TAIL = _EPW - _NCHF * _CHUNK


def _fuse_relu(ra_v, rb_v, ho_v, n):
    def fuse(e, c2):
        for j in range(H // 16):
            ho_v[e, pl.ds(j * 16, 16)] = jnp.maximum(
                ra_v[e, pl.ds(j * 16, 16)]
                + rb_v[e, pl.ds(H + j * 16, 16)], 0.0)
        return c2

    lax.fori_loop(0, n, fuse, 0, unroll=4)


def _make_k2_body(seg):
    def _k2_body(t_hbm, ei_hbm, out_hbm,
                 ia_v, ib_v, ra0_v, rb0_v, ra1_v, rb1_v, ra2_v, rb2_v,
                 ho0_v, ho1_v, ho2_v, sg0, sg1, sg2, so0, so1, so2):
        wid = lax.axis_index("s") * 2 + lax.axis_index("c")
        ebase = seg * _ESEG + wid * _EPW
        obase = wid * _EPW
        ra = [ra0_v, ra1_v, ra2_v]
        rb = [rb0_v, rb1_v, rb2_v]
        ho = [ho0_v, ho1_v, ho2_v]
        sg = [sg0, sg1, sg2]
        so = [so0, so1, so2]

        pltpu.sync_copy(ei_hbm.at[pl.ds(ebase, _EPW)], ia_v)
        pltpu.sync_copy(ei_hbm.at[pl.ds(N_EDGES + ebase, _EPW)], ib_v)

        def start_gather(c, s, n=_CHUNK):
            pltpu.async_copy(
                t_hbm.at[ia_v.at[pl.ds(c * _CHUNK, n)]],
                ra[s].at[pl.ds(0, n), :], sg[s])
            pltpu.async_copy(
                t_hbm.at[ib_v.at[pl.ds(c * _CHUNK, n)]],
                rb[s].at[pl.ds(0, n), :], sg[s])

        def wait_gather(s, n=_CHUNK):
            pltpu.make_async_copy(t_hbm.at[ia_v.at[pl.ds(0, n)]],
                                  ra[s].at[pl.ds(0, n), :], sg[s]).wait()
            pltpu.make_async_copy(t_hbm.at[ib_v.at[pl.ds(0, n)]],
                                  rb[s].at[pl.ds(0, n), :], sg[s]).wait()

        def start_out(c, s):
            pltpu.async_copy(
                ho[s], out_hbm.at[pl.ds(obase + c * _CHUNK, _CHUNK)], so[s])

        def wait_out(s):
            pltpu.make_async_copy(ho[s], out_hbm.at[pl.ds(0, _CHUNK)],
                                  so[s]).wait()

        start_gather(0, 0)
        start_gather(1, 1)

        def triple(i, carry):
            c0 = 3 * i
            for k in range(3):
                start_gather(c0 + k + 2, (k + 2) % 3)
                wait_gather(k)

                @pl.when(i > 0)
                def _():
                    wait_out(k)

                _fuse_relu(ra[k], rb[k], ho[k], _CHUNK)
                start_out(c0 + k, k)
            return carry

        n3 = (_NCHF - 2) // 3
        lax.fori_loop(0, n3, triple, 0)

        for c in range(3 * n3, _NCHF):
            s = c % 3
            if c >= 3 * n3 + 2:
                start_gather(c, s)
            wait_gather(s)
            wait_out(s)
            _fuse_relu(ra[s], rb[s], ho[s], _CHUNK)
            start_out(c, s)

        if _TAIL:
            s = _NCHF % 3
            start_gather(_NCHF, s, _TAIL)
            wait_gather(s, _TAIL)
            wait_out(s)
            _fuse_relu(ra[s], rb[s], ho[s], _TAIL)
            pltpu.sync_copy(
                ho[s].at[pl.ds(0, _TAIL), :],
                out_hbm.at[pl.ds(obase + _NCHF * _CHUNK, _TAIL)])
            rest = [k for k in range(3) if k != s]
        else:
            rest = range(3)
        for k in rest:
            wait_out(k)

    return _k2_body


def _gather_relu(T, edge_index, seg):
    mesh = plsc.VectorSubcoreMesh(core_axis_name="c", subcore_axis_name="s")
    k = functools.partial(
        pl.kernel,
        mesh=mesh,
        out_type=jax.ShapeDtypeStruct((_ESEG, H), jnp.float32),
        scratch_types=(
            [pltpu.VMEM((_EPW,), jnp.int32)] * 2
            + [pltpu.VMEM((_CHUNK, 2 * H), jnp.float32)] * 6
            + [pltpu.VMEM((_CHUNK, H), jnp.float32)] * 3
            + [pltpu.SemaphoreType.DMA] * 6
        ),
    )(_make_k2_body(seg))
    return k(T, edge_index)


_EDGE_BLK = 16000


def _k3_body(h_ref, w2_ref, b2_ref, w3_ref, b3_ref, o_ref):
    h2t = lax.dot_general(w2_ref[...], h_ref[...], (((0,), (1,)), ((), ())),
                          preferred_element_type=jnp.float32)
    h2t = jnp.maximum(h2t + b2_ref[...], 0.0)
    o_ref[...] = lax.dot_general(w3_ref[...], h2t, (((0,), (0,)), ((), ())),
                                 preferred_element_type=jnp.float32) + b3_ref[0]


def _k3_body_seg(h_ref, w2_ref, b2_ref, w3_ref, b3_ref, prev_ref, o_ref):
    del prev_ref
    _k3_body(h_ref, w2_ref, b2_ref, w3_ref, b3_ref, o_ref)


def _mlp_tail(H1, W2, b2, W3, b3, seg, prev):
    n_edges = H1.shape[0]
    n_blocks = n_edges // _EDGE_BLK
    nh = W2.shape[1]
    base = seg * n_blocks
    if prev is None:
        body, extra, aliases = _k3_body, (), {}
    else:
        body, extra = _k3_body_seg, (prev,)
        aliases = {5: 0}
    return pl.pallas_call(
        body,
        grid=(n_blocks,),
        in_specs=[
            pl.BlockSpec((_EDGE_BLK, H), lambda i: (i, 0)),
            pl.BlockSpec((H, nh), lambda i: (0, 0)),
            pl.BlockSpec((nh, 1), lambda i: (0, 0)),
            pl.BlockSpec((nh, 1), lambda i: (0, 0)),
            pl.BlockSpec(memory_space=pltpu.SMEM),
        ] + ([pl.BlockSpec(memory_space=pl.ANY)] if prev is not None
             else []),
        out_specs=pl.BlockSpec((1, _EDGE_BLK), lambda i, base=base: (0, base + i)),
        out_shape=jax.ShapeDtypeStruct((1, N_EDGES), jnp.float32),
        input_output_aliases=aliases,
    )(H1, W2, b2.reshape(nh, 1), W3, b3, *extra)


def kernel(Z, edge_index, node_idx, W1, b1, W2, b2, W3, b3):
    zv = lax.dynamic_slice(Z, (node_idx, 0), (1, D))
    T = _build_table(Z, W1, zv, b1.reshape(1, H))
    ei = edge_index.astype(jnp.int32).reshape(2 * N_EDGES)
    out = None
    for seg in range(_NSEG):
        H1 = _gather_relu(T, ei, seg)
        out = _mlp_tail(H1, W2, b2, W3, b3, seg, out)
    return out.reshape(N_EDGES)
